# Initial kernel scaffold; baseline (speedup 1.0000x reference)
import functools

import jax
import jax.numpy as jnp
from jax import lax
from jax.experimental import pallas as pl
from jax.experimental.pallas import tpu as pltpu
from jax.experimental.pallas import tpu_sc as plsc

N = 10000
E = 320000
NPAD = 10240
CHUNK = 128
TILES = 16
ROWS_PER_TILE = 160
EPAD = TILES * ROWS_PER_TILE * CHUNK
EPT = EPAD // TILES
ZROWS = NPAD // TILES


@functools.lru_cache(maxsize=None)
def _make_sc_scatter(dh):
    mesh = plsc.VectorSubcoreMesh(core_axis_name="c", subcore_axis_name="s")

    @functools.partial(
        pl.kernel,
        mesh=mesh,
        out_type=jax.ShapeDtypeStruct((2, NPAD, dh), jnp.float32),
        scratch_types=[
            pltpu.VMEM((EPT,), jnp.int32),
            pltpu.VMEM((ROWS_PER_TILE, CHUNK), jnp.int32),
            pltpu.VMEM((CHUNK, dh), jnp.float32),
            pltpu.VMEM_SHARED((NPAD, dh), jnp.float32),
            pltpu.SemaphoreType.DMA,
        ],
    )
    def sc_scatter(xcat, srcs, dsts, zeros, out, src_v, dst_v, rows_v, agg_s, sem):
        c = lax.axis_index("c")
        s = lax.axis_index("s")
        pltpu.sync_copy(zeros, agg_s.at[pl.ds(s * ZROWS, ZROWS)])
        pltpu.sync_copy(srcs.at[c, pl.ds(s * EPT, EPT)], src_v)
        pltpu.sync_copy(dsts.at[pl.ds(s * ROWS_PER_TILE, ROWS_PER_TILE)], dst_v)
        plsc.subcore_barrier()

        def body(j, carry):
            off = pl.multiple_of(j * CHUNK, CHUNK)
            pltpu.async_copy(xcat.at[src_v.at[pl.ds(off, CHUNK)]], rows_v, sem).wait()
            pltpu.sync_copy(rows_v, agg_s.at[dst_v.at[j]], add=True)
            return carry

        lax.fori_loop(0, ROWS_PER_TILE, body, 0)
        plsc.subcore_barrier()
        pltpu.sync_copy(agg_s.at[pl.ds(s * ZROWS, ZROWS)],
                        out.at[c, pl.ds(s * ZROWS, ZROWS)])

    return sc_scatter


def _mid_body(x_ref, a_ref, w_ref, b_ref, s_ref, o_ref):
    k2 = x_ref.shape[2]
    sc = s_ref[0, 0]
    s0 = sc * x_ref[0] + a_ref[0]
    s1 = sc * x_ref[1] + a_ref[1]
    w = w_ref[...]
    acc = lax.dot_general(s0, w[:, :k2], (((1,), (1,)), ((), ())),
                          preferred_element_type=jnp.float32)
    acc = acc + lax.dot_general(s1, w[:, k2:], (((1,), (1,)), ((), ())),
                                preferred_element_type=jnp.float32)
    acc = jnp.maximum(acc + b_ref[...], 0.0)
    o_ref[0] = acc[:, :128]
    o_ref[1] = acc[:, 128:]


def _tc_mid(x2, agg2, w, b, scale):
    k2 = x2.shape[2]
    r = 1280
    grid = NPAD // r
    return pl.pallas_call(
        _mid_body,
        grid=(grid,),
        in_specs=[
            pl.BlockSpec((2, r, k2), lambda i: (0, i, 0)),
            pl.BlockSpec((2, r, k2), lambda i: (0, i, 0)),
            pl.BlockSpec((256, 2 * k2), lambda i: (0, 0)),
            pl.BlockSpec((1, 256), lambda i: (0, 0)),
            pl.BlockSpec(memory_space=pltpu.SMEM),
        ],
        out_specs=pl.BlockSpec((2, r, 128), lambda i: (0, i, 0)),
        out_shape=jax.ShapeDtypeStruct((2, NPAD, 128), jnp.float32),
    )(x2, agg2, w, b.reshape(1, 256), scale)


def _last_body(do_abs, x_ref, a_ref, w_ref, b_ref, s_ref, hw_ref, hb_ref, o_ref):
    k2 = x_ref.shape[2]
    sc = s_ref[0, 0]
    s0 = sc * x_ref[0] + a_ref[0]
    s1 = sc * x_ref[1] + a_ref[1]
    w = w_ref[...]
    acc = lax.dot_general(s0, w[:, :k2], (((1,), (1,)), ((), ())),
                          preferred_element_type=jnp.float32)
    acc = acc + lax.dot_general(s1, w[:, k2:], (((1,), (1,)), ((), ())),
                                preferred_element_type=jnp.float32)
    acc = jnp.maximum(acc + b_ref[...], 0.0)
    ho = lax.dot_general(acc, hw_ref[...], (((1,), (1,)), ((), ())),
                         preferred_element_type=jnp.float32) + hb_ref[...]
    o_ref[...] = jnp.abs(ho) if do_abs else ho


def _tc_last(x2, agg2, w, b, scale, head_w, head_b, do_abs):
    k2 = x2.shape[2]
    hw = head_w.shape[0]
    r = 1280
    grid = NPAD // r
    return pl.pallas_call(
        functools.partial(_last_body, do_abs),
        grid=(grid,),
        in_specs=[
            pl.BlockSpec((2, r, k2), lambda i: (0, i, 0)),
            pl.BlockSpec((2, r, k2), lambda i: (0, i, 0)),
            pl.BlockSpec((256, 2 * k2), lambda i: (0, 0)),
            pl.BlockSpec((1, 256), lambda i: (0, 0)),
            pl.BlockSpec(memory_space=pltpu.SMEM),
            pl.BlockSpec((hw, 256), lambda i: (0, 0)),
            pl.BlockSpec((1, hw), lambda i: (0, 0)),
        ],
        out_specs=pl.BlockSpec((r, hw), lambda i: (i, 0)),
        out_shape=jax.ShapeDtypeStruct((NPAD, hw), jnp.float32),
    )(x2, agg2, w, b.reshape(1, 256), scale, head_w, head_b.reshape(1, hw))


def _edge_prep(ei):
    src = ei[0]
    dst = ei[1]
    srcp = jnp.concatenate([src, jnp.zeros((EPAD - E,), jnp.int32)])
    srcs = jnp.stack([srcp, srcp + NPAD])
    dstp = jnp.concatenate([dst, jnp.full((EPAD - E,), N, jnp.int32)])
    return srcs, dstp.reshape(EPAD // CHUNK, CHUNK)


def kernel(high_emb, low_emb, high_edge_index, low_edge_index,
           W0, b0, eps0, W1, b1, eps1, W2, b2, eps2,
           high_W, high_b, low_W, low_b, alpha):
    f32 = jnp.float32
    srcs_h, dst_h = _edge_prep(high_edge_index)
    srcs_l, dst_l = _edge_prep(low_edge_index)
    z64 = jnp.zeros((ZROWS, 64), f32)
    z128 = jnp.zeros((ZROWS, 128), f32)
    sc64 = _make_sc_scatter(64)
    sc128 = _make_sc_scatter(128)

    def planes0(x):
        xp = jnp.pad(x, ((0, NPAD - N), (0, 0)))
        return jnp.stack([xp[:, :64], xp[:, 64:]])

    def run_graph(emb, srcs, dst2, head_w, head_b, do_abs):
        x2 = planes0(emb)
        agg = sc64(x2.reshape(2 * NPAD, 64), srcs, dst2, z64)
        x2 = _tc_mid(x2, agg, W0, b0, (1.0 + eps0).reshape(1, 1))
        agg = sc128(x2.reshape(2 * NPAD, 128), srcs, dst2, z128)
        x2 = _tc_mid(x2, agg, W1, b1, (1.0 + eps1).reshape(1, 1))
        agg = sc128(x2.reshape(2 * NPAD, 128), srcs, dst2, z128)
        out = _tc_last(x2, agg, W2, b2, (1.0 + eps2).reshape(1, 1),
                       head_w, head_b, do_abs)
        return out[:N]

    h_out = run_graph(high_emb, srcs_h, dst_h, high_W, high_b, False)
    l_out = run_graph(low_emb, srcs_l, dst_l, low_W, low_b, True)
    return (h_out, l_out, jax.nn.sigmoid(alpha))


# speedup vs baseline: 1.9288x; 1.9288x over previous
import functools

import jax
import jax.numpy as jnp
from jax import lax
from jax.experimental import pallas as pl
from jax.experimental.pallas import tpu as pltpu
from jax.experimental.pallas import tpu_sc as plsc

N = 10000
E = 320000
NPAD = 10240
CHUNK = 128
TILES = 16
ROWS_PER_TILE = 160
EPAD = TILES * ROWS_PER_TILE * CHUNK
EPT = EPAD // TILES
ZROWS = NPAD // TILES
DH = 128
SB = 16
SBE = SB * CHUNK


def _sc_scatter_planes():
    mesh = plsc.VectorSubcoreMesh(core_axis_name="c", subcore_axis_name="s")

    @functools.partial(
        pl.kernel,
        mesh=mesh,
        out_type=jax.ShapeDtypeStruct((2, NPAD, DH), jnp.float32),
        scratch_types=[
            pltpu.VMEM((SBE,), jnp.int32),
            pltpu.VMEM((SB, CHUNK), jnp.int32),
            pltpu.VMEM((CHUNK, DH), jnp.float32),
            pltpu.VMEM_SHARED((NPAD, DH), jnp.float32),
            pltpu.SemaphoreType.DMA,
        ],
    )
    def sc_scatter(xcat, srcs, dsts, zeros, out, src_v, dst_v, rows_v, agg_s, sem):
        c = lax.axis_index("c")
        s = lax.axis_index("s")
        pltpu.sync_copy(zeros, agg_s.at[pl.ds(s * ZROWS, ZROWS)])
        plsc.subcore_barrier()

        def outer(g, carry):
            eoff = pl.multiple_of(s * EPT + g * SBE, CHUNK)
            roff = pl.multiple_of(s * (EPT // CHUNK) + g * SB, 8)
            pltpu.sync_copy(srcs.at[c, pl.ds(eoff, SBE)], src_v)
            pltpu.sync_copy(dsts.at[pl.ds(roff, SB)], dst_v)

            def body(j, carry2):
                off = pl.multiple_of(j * CHUNK, CHUNK)
                pltpu.async_copy(xcat.at[src_v.at[pl.ds(off, CHUNK)]],
                                 rows_v, sem).wait()
                pltpu.sync_copy(rows_v, agg_s.at[dst_v.at[j]], add=True)
                return carry2

            lax.fori_loop(0, SB, body, 0)
            return carry

        lax.fori_loop(0, EPT // SBE, outer, 0)
        plsc.subcore_barrier()
        pltpu.sync_copy(agg_s.at[pl.ds(s * ZROWS, ZROWS)],
                        out.at[c, pl.ds(s * ZROWS, ZROWS)])

    return sc_scatter


EPT0 = EPAD // 32
RPT0 = EPT0 // CHUNK


def _sc_scatter_edgesplit():
    mesh = plsc.VectorSubcoreMesh(core_axis_name="c", subcore_axis_name="s")

    @functools.partial(
        pl.kernel,
        mesh=mesh,
        out_type=jax.ShapeDtypeStruct((2, NPAD, DH), jnp.float32),
        scratch_types=[
            pltpu.VMEM((SBE,), jnp.int32),
            pltpu.VMEM((SB, CHUNK), jnp.int32),
            pltpu.VMEM((CHUNK, DH), jnp.float32),
            pltpu.VMEM_SHARED((NPAD, DH), jnp.float32),
            pltpu.SemaphoreType.DMA,
        ],
    )
    def sc_scatter(x, srcs, dsts, zeros, out, src_v, dst_v, rows_v, agg_s, sem):
        c = lax.axis_index("c")
        s = lax.axis_index("s")
        w = c * TILES + s
        pltpu.sync_copy(zeros, agg_s.at[pl.ds(s * ZROWS, ZROWS)])
        plsc.subcore_barrier()

        def outer(g, carry):
            eoff = pl.multiple_of(w * EPT0 + g * SBE, CHUNK)
            roff = pl.multiple_of(w * RPT0 + g * SB, 8)
            pltpu.sync_copy(srcs.at[pl.ds(eoff, SBE)], src_v)
            pltpu.sync_copy(dsts.at[pl.ds(roff, SB)], dst_v)

            def body(j, carry2):
                off = pl.multiple_of(j * CHUNK, CHUNK)
                pltpu.async_copy(x.at[src_v.at[pl.ds(off, CHUNK)]],
                                 rows_v, sem).wait()
                pltpu.sync_copy(rows_v, agg_s.at[dst_v.at[j]], add=True)
                return carry2

            lax.fori_loop(0, SB, body, 0)
            return carry

        lax.fori_loop(0, EPT0 // SBE, outer, 0)
        plsc.subcore_barrier()
        pltpu.sync_copy(agg_s.at[pl.ds(s * ZROWS, ZROWS)],
                        out.at[c, pl.ds(s * ZROWS, ZROWS)])

    return sc_scatter


def _l0_body(x_ref, a_ref, w_ref, b_ref, s_ref, o_ref):
    sc = s_ref[0, 0]
    s0 = sc * x_ref[...] + a_ref[0] + a_ref[1]
    acc = lax.dot_general(s0, w_ref[...], (((1,), (1,)), ((), ())),
                          preferred_element_type=jnp.float32)
    acc = jnp.maximum(acc + b_ref[...], 0.0)
    o_ref[0] = acc[:, :128]
    o_ref[1] = acc[:, 128:]


def _tc_l0(x, agg2, w, b, scale):
    r = 1280
    return pl.pallas_call(
        _l0_body,
        grid=(NPAD // r,),
        in_specs=[
            pl.BlockSpec((r, 128), lambda i: (i, 0)),
            pl.BlockSpec((2, r, 128), lambda i: (0, i, 0)),
            pl.BlockSpec((256, 128), lambda i: (0, 0)),
            pl.BlockSpec((1, 256), lambda i: (0, 0)),
            pl.BlockSpec(memory_space=pltpu.SMEM),
        ],
        out_specs=pl.BlockSpec((2, r, 128), lambda i: (0, i, 0)),
        out_shape=jax.ShapeDtypeStruct((2, NPAD, 128), jnp.float32),
    )(x, agg2, w, b.reshape(1, 256), scale)


def _mid_body(x_ref, a_ref, w_ref, b_ref, s_ref, o_ref):
    sc = s_ref[0, 0]
    s0 = sc * x_ref[0] + a_ref[0]
    s1 = sc * x_ref[1] + a_ref[1]
    w = w_ref[...]
    acc = lax.dot_general(s0, w[:, :128], (((1,), (1,)), ((), ())),
                          preferred_element_type=jnp.float32)
    acc = acc + lax.dot_general(s1, w[:, 128:], (((1,), (1,)), ((), ())),
                                preferred_element_type=jnp.float32)
    acc = jnp.maximum(acc + b_ref[...], 0.0)
    o_ref[0] = acc[:, :128]
    o_ref[1] = acc[:, 128:]


def _tc_mid(x2, agg2, w, b, scale):
    r = 1280
    return pl.pallas_call(
        _mid_body,
        grid=(NPAD // r,),
        in_specs=[
            pl.BlockSpec((2, r, 128), lambda i: (0, i, 0)),
            pl.BlockSpec((2, r, 128), lambda i: (0, i, 0)),
            pl.BlockSpec((256, 256), lambda i: (0, 0)),
            pl.BlockSpec((1, 256), lambda i: (0, 0)),
            pl.BlockSpec(memory_space=pltpu.SMEM),
        ],
        out_specs=pl.BlockSpec((2, r, 128), lambda i: (0, i, 0)),
        out_shape=jax.ShapeDtypeStruct((2, NPAD, 128), jnp.float32),
    )(x2, agg2, w, b.reshape(1, 256), scale)


def _last_body(do_abs, x_ref, a_ref, w_ref, b_ref, s_ref, hw_ref, hb_ref, o_ref):
    sc = s_ref[0, 0]
    s0 = sc * x_ref[0] + a_ref[0]
    s1 = sc * x_ref[1] + a_ref[1]
    w = w_ref[...]
    acc = lax.dot_general(s0, w[:, :128], (((1,), (1,)), ((), ())),
                          preferred_element_type=jnp.float32)
    acc = acc + lax.dot_general(s1, w[:, 128:], (((1,), (1,)), ((), ())),
                                preferred_element_type=jnp.float32)
    acc = jnp.maximum(acc + b_ref[...], 0.0)
    ho = lax.dot_general(acc, hw_ref[...], (((1,), (1,)), ((), ())),
                         preferred_element_type=jnp.float32) + hb_ref[...]
    o_ref[...] = jnp.abs(ho) if do_abs else ho


def _tc_last(x2, agg2, w, b, scale, head_w, head_b, do_abs):
    hw = head_w.shape[0]
    hwp = jnp.zeros((128, 256), jnp.float32).at[:hw].set(head_w)
    hbp = jnp.zeros((1, 128), jnp.float32).at[0, :hw].set(head_b)
    r = 1280
    return pl.pallas_call(
        functools.partial(_last_body, do_abs),
        grid=(NPAD // r,),
        in_specs=[
            pl.BlockSpec((2, r, 128), lambda i: (0, i, 0)),
            pl.BlockSpec((2, r, 128), lambda i: (0, i, 0)),
            pl.BlockSpec((256, 256), lambda i: (0, 0)),
            pl.BlockSpec((1, 256), lambda i: (0, 0)),
            pl.BlockSpec(memory_space=pltpu.SMEM),
            pl.BlockSpec((128, 256), lambda i: (0, 0)),
            pl.BlockSpec((1, 128), lambda i: (0, 0)),
        ],
        out_specs=pl.BlockSpec((r, 128), lambda i: (i, 0)),
        out_shape=jax.ShapeDtypeStruct((NPAD, 128), jnp.float32),
    )(x2, agg2, w, b.reshape(1, 256), scale, hwp, hbp)[:, :hw]


def _edge_prep(ei):
    src = ei[0]
    dst = ei[1]
    srcp = jnp.concatenate([src, jnp.zeros((EPAD - E,), jnp.int32)])
    srcs2 = jnp.stack([srcp, srcp + NPAD])
    dstp = jnp.concatenate([dst, jnp.full((EPAD - E,), N, jnp.int32)])
    return srcp, srcs2, dstp.reshape(EPAD // CHUNK, CHUNK)


def kernel(high_emb, low_emb, high_edge_index, low_edge_index,
           W0, b0, eps0, W1, b1, eps1, W2, b2, eps2,
           high_W, high_b, low_W, low_b, alpha):
    f32 = jnp.float32
    src_h, srcs2_h, dst_h = _edge_prep(high_edge_index)
    src_l, srcs2_l, dst_l = _edge_prep(low_edge_index)
    z128 = jnp.zeros((ZROWS, DH), f32)
    sc_es = _sc_scatter_edgesplit()
    sc_pl = _sc_scatter_planes()

    def run_graph(emb, src1, srcs2, dst2, head_w, head_b, do_abs):
        x = jnp.pad(emb, ((0, NPAD - N), (0, 0)))
        agg = sc_es(x, src1, dst2, z128)
        x2 = _tc_l0(x, agg, W0, b0, (1.0 + eps0).reshape(1, 1))
        agg = sc_pl(x2.reshape(2 * NPAD, DH), srcs2, dst2, z128)
        x2 = _tc_mid(x2, agg, W1, b1, (1.0 + eps1).reshape(1, 1))
        agg = sc_pl(x2.reshape(2 * NPAD, DH), srcs2, dst2, z128)
        out = _tc_last(x2, agg, W2, b2, (1.0 + eps2).reshape(1, 1),
                       head_w, head_b, do_abs)
        return out[:N]

    h_out = run_graph(high_emb, src_h, srcs2_h, dst_h, high_W, high_b, False)
    l_out = run_graph(low_emb, src_l, srcs2_l, dst_l, low_W, low_b, True)
    return (h_out, l_out, jax.nn.sigmoid(alpha))


# speedup vs baseline: 2.1612x; 1.1205x over previous
import functools

import jax
import jax.numpy as jnp
from jax import lax
from jax.experimental import pallas as pl
from jax.experimental.pallas import tpu as pltpu
from jax.experimental.pallas import tpu_sc as plsc

N = 10000
E = 320000
NPAD = 10240
CHUNK = 128
TILES = 16
ROWS_PER_TILE = 160
EPAD = TILES * ROWS_PER_TILE * CHUNK
EPT = EPAD // TILES
ZROWS = NPAD // TILES
DH = 128
SB = 16
SBE = SB * CHUNK


def _sc_scatter_planes():
    mesh = plsc.VectorSubcoreMesh(core_axis_name="c", subcore_axis_name="s")

    @functools.partial(
        pl.kernel,
        mesh=mesh,
        out_type=jax.ShapeDtypeStruct((2, NPAD, DH), jnp.float32),
        scratch_types=[
            pltpu.VMEM((SBE,), jnp.int32),
            pltpu.VMEM((SB, CHUNK), jnp.int32),
            pltpu.VMEM((2, CHUNK, DH), jnp.float32),
            pltpu.VMEM_SHARED((NPAD, DH), jnp.float32),
            pltpu.SemaphoreType.DMA,
        ],
    )
    def sc_scatter(xcat, srcs, dsts, zeros, out, src_v, dst_v, rows_v, agg_s, sem):
        c = lax.axis_index("c")
        s = lax.axis_index("s")
        pltpu.sync_copy(zeros, agg_s.at[pl.ds(s * ZROWS, ZROWS)])
        plsc.subcore_barrier()

        def outer(g, carry):
            eoff = pl.multiple_of(s * EPT + g * SBE, CHUNK)
            roff = pl.multiple_of(s * (EPT // CHUNK) + g * SB, 8)
            pltpu.sync_copy(srcs.at[c, pl.ds(eoff, SBE)], src_v)
            pltpu.sync_copy(dsts.at[pl.ds(roff, SB)], dst_v)

            def body(j, carry2):
                b = lax.rem(j, 2)
                off = pl.multiple_of(j * CHUNK, CHUNK)
                cp = pltpu.async_copy(xcat.at[src_v.at[pl.ds(off, CHUNK)]],
                                      rows_v.at[b], sem)

                @pl.when(j > 0)
                def _():
                    pltpu.sync_copy(rows_v.at[1 - b],
                                    agg_s.at[dst_v.at[j - 1]], add=True)

                cp.wait()
                return carry2

            lax.fori_loop(0, SB, body, 0)
            pltpu.sync_copy(rows_v.at[(SB - 1) % 2],
                            agg_s.at[dst_v.at[SB - 1]], add=True)
            return carry

        lax.fori_loop(0, EPT // SBE, outer, 0)
        plsc.subcore_barrier()
        pltpu.sync_copy(agg_s.at[pl.ds(s * ZROWS, ZROWS)],
                        out.at[c, pl.ds(s * ZROWS, ZROWS)])

    return sc_scatter


EPT0 = EPAD // 32
RPT0 = EPT0 // CHUNK


def _sc_scatter_edgesplit():
    mesh = plsc.VectorSubcoreMesh(core_axis_name="c", subcore_axis_name="s")

    @functools.partial(
        pl.kernel,
        mesh=mesh,
        out_type=jax.ShapeDtypeStruct((2, NPAD, DH), jnp.float32),
        scratch_types=[
            pltpu.VMEM((SBE,), jnp.int32),
            pltpu.VMEM((SB, CHUNK), jnp.int32),
            pltpu.VMEM((2, CHUNK, DH), jnp.float32),
            pltpu.VMEM_SHARED((NPAD, DH), jnp.float32),
            pltpu.SemaphoreType.DMA,
        ],
    )
    def sc_scatter(x, srcs, dsts, zeros, out, src_v, dst_v, rows_v, agg_s, sem):
        c = lax.axis_index("c")
        s = lax.axis_index("s")
        w = c * TILES + s
        pltpu.sync_copy(zeros, agg_s.at[pl.ds(s * ZROWS, ZROWS)])
        plsc.subcore_barrier()

        def outer(g, carry):
            eoff = pl.multiple_of(w * EPT0 + g * SBE, CHUNK)
            roff = pl.multiple_of(w * RPT0 + g * SB, 8)
            pltpu.sync_copy(srcs.at[pl.ds(eoff, SBE)], src_v)
            pltpu.sync_copy(dsts.at[pl.ds(roff, SB)], dst_v)

            def body(j, carry2):
                b = lax.rem(j, 2)
                off = pl.multiple_of(j * CHUNK, CHUNK)
                cp = pltpu.async_copy(x.at[src_v.at[pl.ds(off, CHUNK)]],
                                      rows_v.at[b], sem)

                @pl.when(j > 0)
                def _():
                    pltpu.sync_copy(rows_v.at[1 - b],
                                    agg_s.at[dst_v.at[j - 1]], add=True)

                cp.wait()
                return carry2

            lax.fori_loop(0, SB, body, 0)
            pltpu.sync_copy(rows_v.at[(SB - 1) % 2],
                            agg_s.at[dst_v.at[SB - 1]], add=True)
            return carry

        lax.fori_loop(0, EPT0 // SBE, outer, 0)
        plsc.subcore_barrier()
        pltpu.sync_copy(agg_s.at[pl.ds(s * ZROWS, ZROWS)],
                        out.at[c, pl.ds(s * ZROWS, ZROWS)])

    return sc_scatter


def _l0_body(x_ref, a_ref, w_ref, b_ref, s_ref, o_ref):
    sc = s_ref[0, 0]
    s0 = sc * x_ref[...] + a_ref[0] + a_ref[1]
    acc = lax.dot_general(s0, w_ref[...], (((1,), (1,)), ((), ())),
                          preferred_element_type=jnp.float32)
    acc = jnp.maximum(acc + b_ref[...], 0.0)
    o_ref[0] = acc[:, :128]
    o_ref[1] = acc[:, 128:]


def _tc_l0(x, agg2, w, b, scale):
    r = 1280
    return pl.pallas_call(
        _l0_body,
        grid=(NPAD // r,),
        in_specs=[
            pl.BlockSpec((r, 128), lambda i: (i, 0)),
            pl.BlockSpec((2, r, 128), lambda i: (0, i, 0)),
            pl.BlockSpec((256, 128), lambda i: (0, 0)),
            pl.BlockSpec((1, 256), lambda i: (0, 0)),
            pl.BlockSpec(memory_space=pltpu.SMEM),
        ],
        out_specs=pl.BlockSpec((2, r, 128), lambda i: (0, i, 0)),
        out_shape=jax.ShapeDtypeStruct((2, NPAD, 128), jnp.float32),
    )(x, agg2, w, b.reshape(1, 256), scale)


def _mid_body(x_ref, a_ref, w_ref, b_ref, s_ref, o_ref):
    sc = s_ref[0, 0]
    s0 = sc * x_ref[0] + a_ref[0]
    s1 = sc * x_ref[1] + a_ref[1]
    w = w_ref[...]
    acc = lax.dot_general(s0, w[:, :128], (((1,), (1,)), ((), ())),
                          preferred_element_type=jnp.float32)
    acc = acc + lax.dot_general(s1, w[:, 128:], (((1,), (1,)), ((), ())),
                                preferred_element_type=jnp.float32)
    acc = jnp.maximum(acc + b_ref[...], 0.0)
    o_ref[0] = acc[:, :128]
    o_ref[1] = acc[:, 128:]


def _tc_mid(x2, agg2, w, b, scale):
    r = 1280
    return pl.pallas_call(
        _mid_body,
        grid=(NPAD // r,),
        in_specs=[
            pl.BlockSpec((2, r, 128), lambda i: (0, i, 0)),
            pl.BlockSpec((2, r, 128), lambda i: (0, i, 0)),
            pl.BlockSpec((256, 256), lambda i: (0, 0)),
            pl.BlockSpec((1, 256), lambda i: (0, 0)),
            pl.BlockSpec(memory_space=pltpu.SMEM),
        ],
        out_specs=pl.BlockSpec((2, r, 128), lambda i: (0, i, 0)),
        out_shape=jax.ShapeDtypeStruct((2, NPAD, 128), jnp.float32),
    )(x2, agg2, w, b.reshape(1, 256), scale)


def _last_body(do_abs, x_ref, a_ref, w_ref, b_ref, s_ref, hw_ref, hb_ref, o_ref):
    sc = s_ref[0, 0]
    s0 = sc * x_ref[0] + a_ref[0]
    s1 = sc * x_ref[1] + a_ref[1]
    w = w_ref[...]
    acc = lax.dot_general(s0, w[:, :128], (((1,), (1,)), ((), ())),
                          preferred_element_type=jnp.float32)
    acc = acc + lax.dot_general(s1, w[:, 128:], (((1,), (1,)), ((), ())),
                                preferred_element_type=jnp.float32)
    acc = jnp.maximum(acc + b_ref[...], 0.0)
    ho = lax.dot_general(acc, hw_ref[...], (((1,), (1,)), ((), ())),
                         preferred_element_type=jnp.float32) + hb_ref[...]
    o_ref[...] = jnp.abs(ho) if do_abs else ho


def _tc_last(x2, agg2, w, b, scale, head_w, head_b, do_abs):
    hw = head_w.shape[0]
    hwp = jnp.zeros((128, 256), jnp.float32).at[:hw].set(head_w)
    hbp = jnp.zeros((1, 128), jnp.float32).at[0, :hw].set(head_b)
    r = 1280
    return pl.pallas_call(
        functools.partial(_last_body, do_abs),
        grid=(NPAD // r,),
        in_specs=[
            pl.BlockSpec((2, r, 128), lambda i: (0, i, 0)),
            pl.BlockSpec((2, r, 128), lambda i: (0, i, 0)),
            pl.BlockSpec((256, 256), lambda i: (0, 0)),
            pl.BlockSpec((1, 256), lambda i: (0, 0)),
            pl.BlockSpec(memory_space=pltpu.SMEM),
            pl.BlockSpec((128, 256), lambda i: (0, 0)),
            pl.BlockSpec((1, 128), lambda i: (0, 0)),
        ],
        out_specs=pl.BlockSpec((r, 128), lambda i: (i, 0)),
        out_shape=jax.ShapeDtypeStruct((NPAD, 128), jnp.float32),
    )(x2, agg2, w, b.reshape(1, 256), scale, hwp, hbp)[:, :hw]


def _edge_prep(ei):
    src = ei[0]
    dst = ei[1]
    srcp = jnp.concatenate([src, jnp.zeros((EPAD - E,), jnp.int32)])
    srcs2 = jnp.stack([srcp, srcp + NPAD])
    dstp = jnp.concatenate([dst, jnp.full((EPAD - E,), N, jnp.int32)])
    return srcp, srcs2, dstp.reshape(EPAD // CHUNK, CHUNK)


def kernel(high_emb, low_emb, high_edge_index, low_edge_index,
           W0, b0, eps0, W1, b1, eps1, W2, b2, eps2,
           high_W, high_b, low_W, low_b, alpha):
    f32 = jnp.float32
    src_h, srcs2_h, dst_h = _edge_prep(high_edge_index)
    src_l, srcs2_l, dst_l = _edge_prep(low_edge_index)
    z128 = jnp.zeros((ZROWS, DH), f32)
    sc_es = _sc_scatter_edgesplit()
    sc_pl = _sc_scatter_planes()

    def run_graph(emb, src1, srcs2, dst2, head_w, head_b, do_abs):
        x = jnp.pad(emb, ((0, NPAD - N), (0, 0)))
        agg = sc_es(x, src1, dst2, z128)
        x2 = _tc_l0(x, agg, W0, b0, (1.0 + eps0).reshape(1, 1))
        agg = sc_pl(x2.reshape(2 * NPAD, DH), srcs2, dst2, z128)
        x2 = _tc_mid(x2, agg, W1, b1, (1.0 + eps1).reshape(1, 1))
        agg = sc_pl(x2.reshape(2 * NPAD, DH), srcs2, dst2, z128)
        out = _tc_last(x2, agg, W2, b2, (1.0 + eps2).reshape(1, 1),
                       head_w, head_b, do_abs)
        return out[:N]

    h_out = run_graph(high_emb, src_h, srcs2_h, dst_h, high_W, high_b, False)
    l_out = run_graph(low_emb, src_l, srcs2_l, dst_l, low_W, low_b, True)
    return (h_out, l_out, jax.nn.sigmoid(alpha))
